# baseline (speedup 1.0000x reference)
import jax
import jax.numpy as jnp
import numpy as np
from jax import lax
from jax.experimental import pallas as pl
from jax.experimental.pallas import tpu as pltpu
from jax.experimental.pallas import tpu_sc as plsc

_B = 4096
_N_NUM = 10
_N_CAT = 26
_VOCAB = 100000
_D = 32
_SEQ = 1 + _N_NUM + _N_CAT

_NC = 2
_NS = 16
_NW = _NC * _NS

_BPW = _B // _NW
_SB = 32
_NSB = _BPW // _SB
_IDX = _SB * _N_CAT
_ROWS = _SB * _SEQ


def _tokenize_body(tables_hbm, cat_hbm, num_hbm, nk_hbm, bias_hbm, off_hbm,
                   out_hbm,
                   off_v, cidx_v, flat_v, rows_v, out_v, num_v, nk_v, bias_v,
                   sem):
    wid = lax.axis_index("s") * _NC + lax.axis_index("c")
    b0 = wid * _BPW

    pltpu.sync_copy(off_hbm, off_v)
    pltpu.sync_copy(nk_hbm, nk_v)
    pltpu.sync_copy(bias_hbm, bias_v)
    pltpu.sync_copy(num_hbm.at[pl.ds(b0 * _N_NUM, _BPW * _N_NUM)],
                    num_v.at[pl.ds(0, _BPW * _N_NUM)])

    def sub_block(s, carry):
        base_b = b0 + s * _SB
        pltpu.sync_copy(cat_hbm.at[pl.ds(base_b * _N_CAT, _IDX)], cidx_v)

        def add_off(i, c):
            sl = pl.ds(i * 16, 16)
            flat_v[sl] = cidx_v[sl] + off_v[sl]
            return c
        lax.fori_loop(0, _IDX // 16, add_off, 0)

        pltpu.async_copy(tables_hbm.at[flat_v], rows_v, sem).wait()

        def per_b(b2, c):
            orow = b2 * _SEQ
            for col in (0, 16):
                out_v[orow, pl.ds(col, 16)] = bias_v[0, pl.ds(col, 16)]
            nrow = num_v[pl.ds((s * _SB + b2) * _N_NUM, 16)]
            for f in range(_N_NUM):
                sval = nrow[f]
                for col in (0, 16):
                    out_v[orow + 1 + f, pl.ds(col, 16)] = (
                        sval * nk_v[f, pl.ds(col, 16)]
                        + bias_v[1 + f, pl.ds(col, 16)])
            def per_j(j, cc):
                r = b2 * _N_CAT + j
                for col in (0, 16):
                    out_v[orow + 1 + _N_NUM + j, pl.ds(col, 16)] = (
                        rows_v[r, pl.ds(col, 16)]
                        + bias_v[1 + _N_NUM + j, pl.ds(col, 16)])
                return cc
            lax.fori_loop(0, _N_CAT, per_j, 0)
            return c
        lax.fori_loop(0, _SB, per_b, 0)

        pltpu.sync_copy(out_v, out_hbm.at[pl.ds(base_b * _SEQ, _ROWS)])
        return carry

    lax.fori_loop(0, _NSB, sub_block, 0)


@jax.jit
def kernel(numerical, categorical, numerical_kernel, tables, bias_kernel):
    tables_flat = tables.reshape(_N_CAT * _VOCAB, _D)
    cat_flat = categorical.reshape(_B * _N_CAT)
    off = jnp.asarray(np.tile(np.arange(_N_CAT, dtype=np.int32) * _VOCAB, _SB))

    mesh = plsc.VectorSubcoreMesh(core_axis_name="c", subcore_axis_name="s")
    run = pl.kernel(
        _tokenize_body,
        out_type=jax.ShapeDtypeStruct((_B * _SEQ, _D), jnp.float32),
        mesh=mesh,
        compiler_params=pltpu.CompilerParams(use_tc_tiling_on_sc=False),
        scratch_types=[
            pltpu.VMEM((_IDX,), jnp.int32),
            pltpu.VMEM((_IDX,), jnp.int32),
            pltpu.VMEM((_IDX,), jnp.int32),
            pltpu.VMEM((_IDX, _D), jnp.float32),
            pltpu.VMEM((_ROWS, _D), jnp.float32),
            pltpu.VMEM((_BPW * _N_NUM + 16,), jnp.float32),
            pltpu.VMEM((_N_NUM, _D), jnp.float32),
            pltpu.VMEM((_SEQ, _D), jnp.float32),
            pltpu.SemaphoreType.DMA,
        ],
    )
    out = run(tables_flat, cat_flat, numerical.reshape(_B * _N_NUM),
              numerical_kernel,
              bias_kernel, off)
    return out.reshape(_B, _SEQ, _D)


# speedup vs baseline: 7.2056x; 7.2056x over previous
import jax
import jax.numpy as jnp
from jax import lax
from jax.experimental import pallas as pl
from jax.experimental.pallas import tpu as pltpu
from jax.experimental.pallas import tpu_sc as plsc

_B = 4096
_N_NUM = 10
_N_CAT = 26
_VOCAB = 100000
_D = 32
_SEQ = 1 + _N_NUM + _N_CAT

_NC = 2
_NS = 16

_CAT_ROWS = _N_CAT * _D
_AUX_ROWS = (1 + _N_NUM) * _D
_AUX_PER_W = 59
_LANES = 16
_HALF0 = 50048
_HALF1 = _VOCAB - _HALF0
_GRP = _B // _LANES


def _tokenize_body(tables_hbm, cat_hbm, num_hbm, nk_hbm, bias_hbm,
                   out_hbm,
                   idx_v, h0_v, h1_v, v0_v, v1_v, nk_v, bias_v,
                   sh0, sh1, so0, so1):
    wid = lax.axis_index("s") * _NC + lax.axis_index("c")

    pltpu.sync_copy(nk_hbm, nk_v)
    pltpu.sync_copy(bias_hbm, bias_v)
    hbufs = (h0_v, h1_v)
    hsems = (sh0, sh1)
    vbufs = (v0_v, v1_v)
    osems = (so0, so1)

    @pl.when(wid < _N_CAT)
    def _cat_worker():
        j = wid
        pltpu.sync_copy(cat_hbm.at[j], idx_v)
        copies = [None, None]
        ocopies = [None, None]

        def fire(d, h):
            off, size = (0, _HALF0) if h == 0 else (_HALF0, _HALF1)
            copies[h] = pltpu.async_copy(
                tables_hbm.at[j * _D + d].at[pl.ds(off, size)],
                hbufs[h], hsems[h])

        def gather_half(d, h):
            copies[h].wait()
            buf = hbufs[h]
            val = vbufs[d % 2]
            base, size = (0, _HALF0) if h == 0 else (_HALF0, _HALF1)
            bval = bias_v[pl.ds((_N_NUM + 1 + j) * _D + d, _LANES)][0]

            def body(i, c):
                sl = pl.ds(i * _LANES, _LANES)
                raw = idx_v[sl]
                loc = raw - base
                locc = jnp.minimum(jnp.maximum(loc, 0), size - 1)
                g = plsc.load_gather(buf, [locc]) + bval
                if h == 0:
                    val[sl] = jnp.where(loc < _HALF0, g, jnp.float32(0.0))
                else:
                    val[sl] = jnp.where(loc >= 0, g, val[sl])
                return c
            lax.fori_loop(0, _GRP, body, 0)

        fire(0, 0)
        for d in range(_D):
            fire(d, 1)
            if d >= 2:
                ocopies[d % 2].wait()
            gather_half(d, 0)
            if d + 1 < _D:
                fire(d + 1, 0)
            gather_half(d, 1)
            ocopies[d % 2] = pltpu.async_copy(
                vbufs[d % 2], out_hbm.at[(_N_NUM + 1 + j) * _D + d],
                osems[d % 2])
        ocopies[0].wait()
        ocopies[1].wait()

    @pl.when(wid >= _N_CAT)
    def _aux_worker():
        aw = wid - _N_CAT
        copies = [None, None]
        ocopies = [None, None]

        def run_idx(m):
            return jnp.minimum(aw * _AUX_PER_W + m, _AUX_ROWS - 1)

        def fire(m):
            s = run_idx(m) // _D
            f_safe = jnp.maximum(s - 1, 0)
            copies[m % 2] = pltpu.async_copy(
                num_hbm.at[f_safe],
                hbufs[m % 2].at[pl.ds(0, _B)], hsems[m % 2])

        fire(0)
        for m in range(_AUX_PER_W):
            a = run_idx(m)
            if m + 1 < _AUX_PER_W:
                fire(m + 1)
            s = a // _D
            d = a % _D
            f_safe = jnp.maximum(s - 1, 0)
            nk_raw = nk_v[pl.ds(f_safe * _D + d, _LANES)][0]
            nkval = jnp.where(s == 0, jnp.float32(0.0), nk_raw)
            bval = bias_v[pl.ds(s * _D + d, _LANES)][0]
            copies[m % 2].wait()
            if m >= 2:
                ocopies[m % 2].wait()
            src = hbufs[m % 2]
            val = vbufs[m % 2]

            def fma(i, c):
                sl = pl.ds(i * _LANES, _LANES)
                val[sl] = src[sl] * nkval + bval
                return c
            lax.fori_loop(0, _GRP, fma, 0)
            ocopies[m % 2] = pltpu.async_copy(
                val, out_hbm.at[a], osems[m % 2])
        ocopies[0].wait()
        ocopies[1].wait()


@jax.jit
def kernel(numerical, categorical, numerical_kernel, tables, bias_kernel):
    tables_t = tables.transpose(0, 2, 1).reshape(_CAT_ROWS, _VOCAB)
    cat_t = categorical.T
    num_t = numerical.T
    nk_flat = jnp.pad(numerical_kernel.reshape(_N_NUM * _D), (0, _LANES))
    bias_flat = jnp.pad(bias_kernel.reshape(_SEQ * _D), (0, _LANES))

    mesh = plsc.VectorSubcoreMesh(core_axis_name="c", subcore_axis_name="s")
    run = pl.kernel(
        _tokenize_body,
        out_type=jax.ShapeDtypeStruct((_SEQ * _D, _B), jnp.float32),
        mesh=mesh,
        compiler_params=pltpu.CompilerParams(needs_layout_passes=False),
        scratch_types=[
            pltpu.VMEM((_B,), jnp.int32),
            pltpu.VMEM((_HALF0,), jnp.float32),
            pltpu.VMEM((_HALF1,), jnp.float32),
            pltpu.VMEM((_B,), jnp.float32),
            pltpu.VMEM((_B,), jnp.float32),
            pltpu.VMEM((_N_NUM * _D + _LANES,), jnp.float32),
            pltpu.VMEM((_SEQ * _D + _LANES,), jnp.float32),
            pltpu.SemaphoreType.DMA,
            pltpu.SemaphoreType.DMA,
            pltpu.SemaphoreType.DMA,
            pltpu.SemaphoreType.DMA,
        ],
    )
    out_t = run(tables_t, cat_t, num_t, nk_flat, bias_flat)
    return out_t.reshape(_SEQ, _D, _B).transpose(2, 0, 1)
